# baseline (speedup 1.0000x reference)
import functools

import jax
import jax.numpy as jnp
from jax.experimental import pallas as pl

B = 4096
IN_DIM = 1024
E_DIM = 128
K = 256
NQ = 4
BETA = 0.25

ROW_BLK = 256


def _zsq_like_reference_t(vt):
    sq = vt * vt
    acc = sq[0:8, :]
    for g in range(1, 16):
        acc = acc + sq[8 * g:8 * g + 8, :]
    acc = acc[0:4, :] + acc[4:8, :]
    acc = acc[0:2, :] + acc[2:4, :]
    return acc[0:1, :] + acc[1:2, :]


def _fused_kernel(x_ref,
                  ew0, ew1, ew2, eb0, eb1, eb2,
                  cbp0, cbp1, cbp2, cbp3,
                  cbt0, cbt1, cbt2, cbt3,
                  esq0, esq1, esq2, esq3,
                  dw0, dw1, dw2, db0, db1, db2,
                  out_ref, loss_ref, idx_ref, oh_ref, logit_ref):
    f32 = jnp.float32
    dot = functools.partial(jnp.dot, precision=jax.lax.Precision.DEFAULT,
                            preferred_element_type=f32)

    h = x_ref[...]
    h = jnp.maximum(dot(h, ew0[...]) + eb0[...], 0.0)
    h = jnp.maximum(dot(h, ew1[...]) + eb1[...], 0.0)
    latent = dot(h, ew2[...]) + eb2[...]

    residual = latent
    xq = jnp.zeros_like(latent)
    loss_sq = jnp.zeros((ROW_BLK, E_DIM), f32)
    lane_iota = jax.lax.broadcasted_iota(jnp.int32, (ROW_BLK, K), 1)
    idx_cols = []
    d_levels = []
    for q, (cbp_ref, cbt_ref, esq_ref) in enumerate(
            zip((cbp0, cbp1, cbp2, cbp3), (cbt0, cbt1, cbt2, cbt3),
                (esq0, esq1, esq2, esq3))):
        cbt = cbt_ref[...]
        esq = esq_ref[...]
        zsq = jnp.transpose(
            _zsq_like_reference_t(jnp.transpose(residual)))
        d = (zsq + esq) - 2.0 * dot(residual, cbt)
        dmin = jnp.min(d, axis=1, keepdims=True)
        idx = jnp.min(jnp.where(d == dmin, lane_iota, K), axis=1,
                      keepdims=True)
        oh = (lane_iota == idx).astype(f32)
        xr = jnp.dot(oh, jnp.transpose(cbt),
                     precision=jax.lax.Precision.HIGHEST,
                     preferred_element_type=f32)
        diff = xr - residual
        loss_sq = loss_sq + diff * diff
        d_levels.append(d)
        idx_cols.append(idx)
        residual = residual - xr
        xq = xq + xr

    idx_all = jnp.concatenate(idx_cols, axis=1)
    idx_ref[...] = idx_all
    oh_ref[...] = (jax.lax.broadcasted_iota(jnp.int32, (ROW_BLK, NQ, K), 2)
                   == idx_all[:, :, None]).astype(f32)
    logit_ref[...] = jnp.stack(d_levels, axis=1)

    h = jnp.maximum(dot(xq, dw0[...]) + db0[...], 0.0)
    h = jnp.maximum(dot(h, dw1[...]) + db1[...], 0.0)
    out_ref[...] = dot(h, dw2[...]) + db2[...]

    scale = (1.0 + BETA) / (NQ * B * E_DIM)
    loss_blk = jnp.sum(loss_sq, keepdims=True)[0:1, 0:1] * scale

    @pl.when(pl.program_id(0) == 0)
    def _():
        loss_ref[...] = jnp.zeros_like(loss_ref)

    loss_ref[...] = loss_ref[...] + loss_blk


def _bf16_planes(cb):
    hi = (cb.astype(jnp.bfloat16)).astype(jnp.float32)
    rem = cb - hi
    mid = (rem.astype(jnp.bfloat16)).astype(jnp.float32)
    lo = ((rem - mid).astype(jnp.bfloat16)).astype(jnp.float32)
    return hi, mid, lo


def kernel(x, enc_Ws, enc_bs, codebooks, dec_Ws, dec_bs):
    grid = (B // ROW_BLK,)

    full = lambda shape: pl.BlockSpec(shape, lambda i: (0,) * len(shape))
    row = lambda shape: pl.BlockSpec((ROW_BLK,) + shape[1:],
                                     lambda i: (i,) + (0,) * (len(shape) - 1))

    in_specs = (
        [row((B, IN_DIM))]
        + [full(W.shape) for W in enc_Ws]
        + [full((1, b.shape[0])) for b in enc_bs]
        + [full((K, 3 * E_DIM))] * NQ
        + [full((E_DIM, K))] * NQ
        + [full((1, K))] * NQ
        + [full(W.shape) for W in dec_Ws]
        + [full((1, b.shape[0])) for b in dec_bs]
    )
    out_shapes = (
        jax.ShapeDtypeStruct((B, IN_DIM), jnp.float32),
        jax.ShapeDtypeStruct((1, 1), jnp.float32),
        jax.ShapeDtypeStruct((B, NQ), jnp.int32),
        jax.ShapeDtypeStruct((B, NQ, K), jnp.float32),
        jax.ShapeDtypeStruct((B, NQ, K), jnp.float32),
    )
    out_specs = (
        row((B, IN_DIM)),
        full((1, 1)),
        row((B, NQ)),
        row((B, NQ, K)),
        row((B, NQ, K)),
    )

    esqs = [jnp.sum(cb ** 2, axis=1)[None, :] for cb in codebooks]
    planes = [jnp.concatenate(_bf16_planes(cb), axis=1).astype(jnp.bfloat16)
              for cb in codebooks]
    args = ([x] + list(enc_Ws) + [b[None, :] for b in enc_bs]
            + planes
            + [cb.T for cb in codebooks] + esqs
            + list(dec_Ws) + [b[None, :] for b in dec_bs])

    out, loss, idxs, ohs, logits = pl.pallas_call(
        _fused_kernel,
        grid=grid,
        in_specs=in_specs,
        out_specs=out_specs,
        out_shape=out_shapes,
    )(*args)

    return (out, jnp.reshape(loss, ()), idxs, ohs, logits)


# speedup vs baseline: 1.2004x; 1.2004x over previous
import functools

import jax
import jax.numpy as jnp
from jax.experimental import pallas as pl

B = 4096
IN_DIM = 1024
E_DIM = 128
K = 256
NQ = 4
BETA = 0.25

ROW_BLK = 1024


def _zsq_like_reference_t(vt):
    sq = vt * vt
    acc = sq[0:8, :]
    for g in range(1, 16):
        acc = acc + sq[8 * g:8 * g + 8, :]
    acc = acc[0:4, :] + acc[4:8, :]
    acc = acc[0:2, :] + acc[2:4, :]
    return acc[0:1, :] + acc[1:2, :]


def _fused_kernel(x_ref,
                  ew0, ew1, ew2, eb0, eb1, eb2,
                  cbp0, cbp1, cbp2, cbp3,
                  cbt0, cbt1, cbt2, cbt3,
                  esq0, esq1, esq2, esq3,
                  dw0, dw1, dw2, db0, db1, db2,
                  out_ref, loss_ref, idx_ref, oh_ref, logit_ref):
    f32 = jnp.float32
    dot = functools.partial(jnp.dot, precision=jax.lax.Precision.DEFAULT,
                            preferred_element_type=f32)

    h = x_ref[...]
    h = jnp.maximum(dot(h, ew0[...]) + eb0[...], 0.0)
    h = jnp.maximum(dot(h, ew1[...]) + eb1[...], 0.0)
    latent = dot(h, ew2[...]) + eb2[...]

    residual = latent
    xq = jnp.zeros_like(latent)
    loss_sq = jnp.zeros((ROW_BLK, E_DIM), f32)
    lane_iota = jax.lax.broadcasted_iota(jnp.int32, (ROW_BLK, K), 1)
    idx_cols = []
    d_levels = []
    for q, (cbp_ref, cbt_ref, esq_ref) in enumerate(
            zip((cbp0, cbp1, cbp2, cbp3), (cbt0, cbt1, cbt2, cbt3),
                (esq0, esq1, esq2, esq3))):
        cbt = cbt_ref[...]
        esq = esq_ref[...]
        zsq = jnp.transpose(
            _zsq_like_reference_t(jnp.transpose(residual)))
        d = (zsq + esq) - 2.0 * dot(residual, cbt)
        dmin = jnp.min(d, axis=1, keepdims=True)
        idx = jnp.min(jnp.where(d == dmin, lane_iota, K), axis=1,
                      keepdims=True)
        oh = (lane_iota == idx).astype(f32)
        xr = jnp.dot(oh, jnp.transpose(cbt),
                     precision=jax.lax.Precision.HIGHEST,
                     preferred_element_type=f32)
        diff = xr - residual
        loss_sq = loss_sq + diff * diff
        d_levels.append(d)
        idx_cols.append(idx)
        residual = residual - xr
        xq = xq + xr

    idx_all = jnp.concatenate(idx_cols, axis=1)
    idx_ref[...] = idx_all
    oh_ref[...] = (jax.lax.broadcasted_iota(jnp.int32, (ROW_BLK, NQ, K), 2)
                   == idx_all[:, :, None]).astype(f32)
    logit_ref[...] = jnp.stack(d_levels, axis=1)

    h = jnp.maximum(dot(xq, dw0[...]) + db0[...], 0.0)
    h = jnp.maximum(dot(h, dw1[...]) + db1[...], 0.0)
    out_ref[...] = dot(h, dw2[...]) + db2[...]

    scale = (1.0 + BETA) / (NQ * B * E_DIM)
    loss_blk = jnp.sum(loss_sq, keepdims=True)[0:1, 0:1] * scale

    @pl.when(pl.program_id(0) == 0)
    def _():
        loss_ref[...] = jnp.zeros_like(loss_ref)

    loss_ref[...] = loss_ref[...] + loss_blk


def _bf16_planes(cb):
    hi = (cb.astype(jnp.bfloat16)).astype(jnp.float32)
    rem = cb - hi
    mid = (rem.astype(jnp.bfloat16)).astype(jnp.float32)
    lo = ((rem - mid).astype(jnp.bfloat16)).astype(jnp.float32)
    return hi, mid, lo


def kernel(x, enc_Ws, enc_bs, codebooks, dec_Ws, dec_bs):
    grid = (B // ROW_BLK,)

    full = lambda shape: pl.BlockSpec(shape, lambda i: (0,) * len(shape))
    row = lambda shape: pl.BlockSpec((ROW_BLK,) + shape[1:],
                                     lambda i: (i,) + (0,) * (len(shape) - 1))

    in_specs = (
        [row((B, IN_DIM))]
        + [full(W.shape) for W in enc_Ws]
        + [full((1, b.shape[0])) for b in enc_bs]
        + [full((K, 3 * E_DIM))] * NQ
        + [full((E_DIM, K))] * NQ
        + [full((1, K))] * NQ
        + [full(W.shape) for W in dec_Ws]
        + [full((1, b.shape[0])) for b in dec_bs]
    )
    out_shapes = (
        jax.ShapeDtypeStruct((B, IN_DIM), jnp.float32),
        jax.ShapeDtypeStruct((1, 1), jnp.float32),
        jax.ShapeDtypeStruct((B, NQ), jnp.int32),
        jax.ShapeDtypeStruct((B, NQ, K), jnp.float32),
        jax.ShapeDtypeStruct((B, NQ, K), jnp.float32),
    )
    out_specs = (
        row((B, IN_DIM)),
        full((1, 1)),
        row((B, NQ)),
        row((B, NQ, K)),
        row((B, NQ, K)),
    )

    esqs = [jnp.sum(cb ** 2, axis=1)[None, :] for cb in codebooks]
    planes = [jnp.concatenate(_bf16_planes(cb), axis=1).astype(jnp.bfloat16)
              for cb in codebooks]
    args = ([x] + list(enc_Ws) + [b[None, :] for b in enc_bs]
            + planes
            + [cb.T for cb in codebooks] + esqs
            + list(dec_Ws) + [b[None, :] for b in dec_bs])

    out, loss, idxs, ohs, logits = pl.pallas_call(
        _fused_kernel,
        grid=grid,
        in_specs=in_specs,
        out_specs=out_specs,
        out_shape=out_shapes,
    )(*args)

    return (out, jnp.reshape(loss, ()), idxs, ohs, logits)


# speedup vs baseline: 1.2601x; 1.0497x over previous
import functools

import jax
import jax.numpy as jnp
from jax.experimental import pallas as pl

B = 4096
IN_DIM = 1024
E_DIM = 128
K = 256
NQ = 4
BETA = 0.25

ROW_BLK = 1024


def _zsq_like_reference_t(vt):
    sq = vt * vt
    acc = sq[0:8, :]
    for g in range(1, 16):
        acc = acc + sq[8 * g:8 * g + 8, :]
    acc = acc[0:4, :] + acc[4:8, :]
    acc = acc[0:2, :] + acc[2:4, :]
    return acc[0:1, :] + acc[1:2, :]


def _fused_kernel(x_ref,
                  ew0, ew1, ew2, eb0, eb1, eb2,
                  cbp0, cbp1, cbp2, cbp3,
                  cbt0, cbt1, cbt2, cbt3,
                  esq0, esq1, esq2, esq3,
                  dw0, dw1, dw2, db0, db1, db2,
                  out_ref, loss_ref, idx_ref, oh_ref, logit_ref):
    f32 = jnp.float32
    dot = functools.partial(jnp.dot, precision=jax.lax.Precision.DEFAULT,
                            preferred_element_type=f32)

    h = x_ref[...]
    h = jnp.maximum(dot(h, ew0[...]) + eb0[...], 0.0)
    h = jnp.maximum(dot(h, ew1[...]) + eb1[...], 0.0)
    latent = dot(h, ew2[...]) + eb2[...]

    residual = latent
    xq = jnp.zeros_like(latent)
    loss_sq = jnp.zeros((ROW_BLK, E_DIM), f32)
    lane_iota = jax.lax.broadcasted_iota(jnp.int32, (ROW_BLK, K), 1)
    idx_cols = []
    for q, (cbp_ref, cbt_ref, esq_ref) in enumerate(
            zip((cbp0, cbp1, cbp2, cbp3), (cbt0, cbt1, cbt2, cbt3),
                (esq0, esq1, esq2, esq3))):
        cbt = cbt_ref[...]
        esq = esq_ref[...]
        zsq = jnp.transpose(
            _zsq_like_reference_t(jnp.transpose(residual)))
        d = (zsq + esq) - 2.0 * dot(residual, cbt)
        dmin = jnp.min(d, axis=1, keepdims=True)
        idx = jnp.min(jnp.where(d == dmin, lane_iota, K), axis=1,
                      keepdims=True)
        oh = (lane_iota == idx).astype(f32)
        xr = jnp.dot(oh, jnp.transpose(cbt),
                     precision=jax.lax.Precision.HIGHEST,
                     preferred_element_type=f32)
        diff = xr - residual
        loss_sq = loss_sq + diff * diff
        logit_ref[:, q, :] = d
        idx_cols.append(idx)
        residual = residual - xr
        xq = xq + xr

    idx_all = jnp.concatenate(idx_cols, axis=1)
    idx_ref[...] = idx_all
    oh_ref[...] = (jax.lax.broadcasted_iota(jnp.int32, (ROW_BLK, NQ, K), 2)
                   == idx_all[:, :, None]).astype(f32)

    h = jnp.maximum(dot(xq, dw0[...]) + db0[...], 0.0)
    h = jnp.maximum(dot(h, dw1[...]) + db1[...], 0.0)
    out_ref[...] = dot(h, dw2[...]) + db2[...]

    scale = (1.0 + BETA) / (NQ * B * E_DIM)
    loss_blk = jnp.sum(loss_sq, keepdims=True)[0:1, 0:1] * scale

    @pl.when(pl.program_id(0) == 0)
    def _():
        loss_ref[...] = jnp.zeros_like(loss_ref)

    loss_ref[...] = loss_ref[...] + loss_blk


def _bf16_planes(cb):
    hi = (cb.astype(jnp.bfloat16)).astype(jnp.float32)
    rem = cb - hi
    mid = (rem.astype(jnp.bfloat16)).astype(jnp.float32)
    lo = ((rem - mid).astype(jnp.bfloat16)).astype(jnp.float32)
    return hi, mid, lo


def kernel(x, enc_Ws, enc_bs, codebooks, dec_Ws, dec_bs):
    grid = (B // ROW_BLK,)

    full = lambda shape: pl.BlockSpec(shape, lambda i: (0,) * len(shape))
    row = lambda shape: pl.BlockSpec((ROW_BLK,) + shape[1:],
                                     lambda i: (i,) + (0,) * (len(shape) - 1))

    in_specs = (
        [row((B, IN_DIM))]
        + [full(W.shape) for W in enc_Ws]
        + [full((1, b.shape[0])) for b in enc_bs]
        + [full((K, 3 * E_DIM))] * NQ
        + [full((E_DIM, K))] * NQ
        + [full((1, K))] * NQ
        + [full(W.shape) for W in dec_Ws]
        + [full((1, b.shape[0])) for b in dec_bs]
    )
    out_shapes = (
        jax.ShapeDtypeStruct((B, IN_DIM), jnp.float32),
        jax.ShapeDtypeStruct((1, 1), jnp.float32),
        jax.ShapeDtypeStruct((B, NQ), jnp.int32),
        jax.ShapeDtypeStruct((B, NQ, K), jnp.float32),
        jax.ShapeDtypeStruct((B, NQ, K), jnp.float32),
    )
    out_specs = (
        row((B, IN_DIM)),
        full((1, 1)),
        row((B, NQ)),
        row((B, NQ, K)),
        row((B, NQ, K)),
    )

    esqs = [jnp.sum(cb ** 2, axis=1)[None, :] for cb in codebooks]
    planes = [jnp.concatenate(_bf16_planes(cb), axis=1).astype(jnp.bfloat16)
              for cb in codebooks]
    args = ([x] + list(enc_Ws) + [b[None, :] for b in enc_bs]
            + planes
            + [cb.T for cb in codebooks] + esqs
            + list(dec_Ws) + [b[None, :] for b in dec_bs])

    out, loss, idxs, ohs, logits = pl.pallas_call(
        _fused_kernel,
        grid=grid,
        in_specs=in_specs,
        out_specs=out_specs,
        out_shape=out_shapes,
    )(*args)

    return (out, jnp.reshape(loss, ()), idxs, ohs, logits)


# speedup vs baseline: 1.2905x; 1.0241x over previous
import functools

import jax
import jax.numpy as jnp
from jax.experimental import pallas as pl
from jax.experimental.pallas import tpu as pltpu

B = 4096
IN_DIM = 1024
E_DIM = 128
K = 256
NQ = 4
BETA = 0.25

ROW_BLK = 1024


def _zsq_like_reference_t(vt):
    sq = vt * vt
    acc = sq[0:8, :]
    for g in range(1, 16):
        acc = acc + sq[8 * g:8 * g + 8, :]
    acc = acc[0:4, :] + acc[4:8, :]
    acc = acc[0:2, :] + acc[2:4, :]
    return acc[0:1, :] + acc[1:2, :]


def _fused_kernel(x_ref,
                  ew0, ew1, ew2, eb0, eb1, eb2,
                  cbt0, cbt1, cbt2, cbt3,
                  esq0, esq1, esq2, esq3,
                  dw0, dw1, dw2, db0, db1, db2,
                  out_ref, loss_ref, idx_ref, oh_ref, logit_ref):
    f32 = jnp.float32
    dot = functools.partial(jnp.dot, precision=jax.lax.Precision.DEFAULT,
                            preferred_element_type=f32)

    h = x_ref[...]
    h = jnp.maximum(dot(h, ew0[...]) + eb0[...], 0.0)
    h = jnp.maximum(dot(h, ew1[...]) + eb1[...], 0.0)
    latent = dot(h, ew2[...]) + eb2[...]

    residual = latent
    xq = jnp.zeros_like(latent)
    loss_sq = jnp.zeros((ROW_BLK, E_DIM), f32)
    lane_iota = jax.lax.broadcasted_iota(jnp.int32, (ROW_BLK, K), 1)
    idx_cols = []
    for q, (cbt_ref, esq_ref) in enumerate(
            zip((cbt0, cbt1, cbt2, cbt3), (esq0, esq1, esq2, esq3))):
        cbt = cbt_ref[...]
        esq = esq_ref[...]
        zsq = jnp.transpose(
            _zsq_like_reference_t(jnp.transpose(residual)))
        d = (zsq + esq) - 2.0 * dot(residual, cbt)
        dmin = jnp.min(d, axis=1, keepdims=True)
        idx = jnp.min(jnp.where(d == dmin, lane_iota, K), axis=1,
                      keepdims=True)
        oh = (lane_iota == idx).astype(f32)
        xr = jnp.dot(oh, jnp.transpose(cbt),
                     precision=jax.lax.Precision.HIGHEST,
                     preferred_element_type=f32)
        diff = xr - residual
        loss_sq = loss_sq + diff * diff
        logit_ref[:, q, :] = d
        idx_cols.append(idx)
        residual = residual - xr
        xq = xq + xr

    idx_all = jnp.concatenate(idx_cols, axis=1)
    idx_ref[...] = idx_all
    oh_ref[...] = (jax.lax.broadcasted_iota(jnp.int32, (ROW_BLK, NQ, K), 2)
                   == idx_all[:, :, None]).astype(f32)

    h = jnp.maximum(dot(xq, dw0[...]) + db0[...], 0.0)
    h = jnp.maximum(dot(h, dw1[...]) + db1[...], 0.0)
    out_ref[...] = dot(h, dw2[...]) + db2[...]

    scale = (1.0 + BETA) / (NQ * B * E_DIM)
    loss_blk = jnp.sum(loss_sq, keepdims=True)[0:1, 0:1] * scale

    @pl.when(pl.program_id(0) == 0)
    def _():
        loss_ref[...] = jnp.zeros_like(loss_ref)

    loss_ref[...] = loss_ref[...] + loss_blk


def kernel(x, enc_Ws, enc_bs, codebooks, dec_Ws, dec_bs):
    grid = (B // ROW_BLK,)

    full = lambda shape: pl.BlockSpec(shape, lambda i: (0,) * len(shape))
    row = lambda shape: pl.BlockSpec((ROW_BLK,) + shape[1:],
                                     lambda i: (i,) + (0,) * (len(shape) - 1))

    in_specs = (
        [row((B, IN_DIM))]
        + [full(W.shape) for W in enc_Ws]
        + [full((1, b.shape[0])) for b in enc_bs]
        + [full((E_DIM, K))] * NQ
        + [full((1, K))] * NQ
        + [full(W.shape) for W in dec_Ws]
        + [full((1, b.shape[0])) for b in dec_bs]
    )
    out_shapes = (
        jax.ShapeDtypeStruct((B, IN_DIM), jnp.float32),
        jax.ShapeDtypeStruct((1, 1), jnp.float32),
        jax.ShapeDtypeStruct((B, NQ), jnp.int32),
        jax.ShapeDtypeStruct((B, NQ, K), jnp.float32),
        jax.ShapeDtypeStruct((B, NQ, K), jnp.float32),
    )
    out_specs = (
        row((B, IN_DIM)),
        full((1, 1)),
        row((B, NQ)),
        row((B, NQ, K)),
        row((B, NQ, K)),
    )

    esqs = [jnp.sum(cb ** 2, axis=1)[None, :] for cb in codebooks]
    args = ([x] + list(enc_Ws) + [b[None, :] for b in enc_bs]
            + [cb.T for cb in codebooks] + esqs
            + list(dec_Ws) + [b[None, :] for b in dec_bs])

    out, loss, idxs, ohs, logits = pl.pallas_call(
        _fused_kernel,
        grid=grid,
        in_specs=in_specs,
        out_specs=out_specs,
        out_shape=out_shapes,
        compiler_params=pltpu.CompilerParams(
            vmem_limit_bytes=128 * 1024 * 1024),
    )(*args)

    return (out, jnp.reshape(loss, ()), idxs, ohs, logits)
